# Initial kernel scaffold; baseline (speedup 1.0000x reference)
import jax
import jax.numpy as jnp
from jax.experimental import pallas as pl


def kernel(x, edge_index, params):
    raise NotImplementedError("write your pallas kernel here")


# speedup vs baseline: 2.7624x; 2.7624x over previous
import functools

import jax
import jax.numpy as jnp
from jax import lax
from jax.experimental import pallas as pl
from jax.experimental.pallas import tpu as pltpu
from jax.experimental.pallas import tpu_sc as plsc

N_NODES = 50000
N_EDGES = 800000
EPS = 1e-5
NPAD = 50048

NC, NS = 2, 16
NW = NC * NS
EPT = N_EDGES // NW
CH = 1000
NCH = EPT // CH
RZT = NPAD // NS


def _sub_pieces(ch):
  pieces = [ch] * (RZT // ch)
  if RZT % ch:
    pieces.append(RZT % ch)
  return tuple(pieces)

BLK_E = 8000
GRID_E = N_EDGES // BLK_E
BLK_N = 6256
GRID_N = NPAD // BLK_N
BLK_X = 10000
GRID_X = N_NODES // BLK_X

_MESH = plsc.VectorSubcoreMesh(core_axis_name="c", subcore_axis_name="s")
_SC_PARAMS = pltpu.CompilerParams(use_tc_tiling_on_sc=False)
F32 = jnp.float32


def _gather_body(h_hbm, src_hbm, dst_hbm, xi_hbm, xj_hbm, idx_d, idx_s,
                 bi, bj, sem):
  wid = lax.axis_index("s") * NC + lax.axis_index("c")
  base_t = wid * EPT

  def chunk(k, carry):
    base = base_t + k * CH
    pltpu.sync_copy(dst_hbm.at[pl.ds(base, CH)], idx_d)
    pltpu.sync_copy(src_hbm.at[pl.ds(base, CH)], idx_s)
    pltpu.async_copy(h_hbm.at[idx_d], bi, sem).wait()
    pltpu.async_copy(h_hbm.at[idx_s], bj, sem).wait()
    pltpu.sync_copy(bi, xi_hbm.at[pl.ds(base, CH)])
    pltpu.sync_copy(bj, xj_hbm.at[pl.ds(base, CH)])
    return carry

  lax.fori_loop(0, NCH, chunk, 0)


def _edge_gather(h):
  w = h.shape[1]
  k = pl.kernel(
      _gather_body,
      out_type=[
          jax.ShapeDtypeStruct((N_EDGES, w), F32),
          jax.ShapeDtypeStruct((N_EDGES, w), F32),
      ],
      mesh=_MESH,
      scratch_types=[
          pltpu.VMEM((CH,), jnp.int32),
          pltpu.VMEM((CH,), jnp.int32),
          pltpu.VMEM((CH, w), F32),
          pltpu.VMEM((CH, w), F32),
          pltpu.SemaphoreType.DMA,
      ],
      compiler_params=_SC_PARAMS,
  )
  return k


def _scatter_body(D, ch, counts, msg_hbm, dst_hbm, zeros_hbm, out_hbm, shared,
                  idx_v, ybuf):
  c = lax.axis_index("c")
  s = lax.axis_index("s")
  wid = s * NC + c
  r0 = s * RZT
  nch = EPT // ch

  pltpu.sync_copy(zeros_hbm, ybuf)
  off = 0
  for sz in _sub_pieces(ch):
    pltpu.sync_copy(ybuf.at[pl.ds(0, sz)], shared.at[pl.ds(r0 + off, sz)])
    off += sz
  plsc.subcore_barrier()

  if counts:
    pltpu.sync_copy(msg_hbm, ybuf)

    def chunk(k, carry):
      base = wid * EPT + k * ch
      pltpu.sync_copy(dst_hbm.at[pl.ds(base, ch)], idx_v)
      pltpu.sync_copy(ybuf, shared.at[idx_v], add=True)
      return carry
  else:

    def chunk(k, carry):
      base = wid * EPT + k * ch
      pltpu.sync_copy(dst_hbm.at[pl.ds(base, ch)], idx_v)
      pltpu.sync_copy(msg_hbm.at[pl.ds(base, ch)], ybuf)
      pltpu.sync_copy(ybuf, shared.at[idx_v], add=True)
      return carry

  lax.fori_loop(0, nch, chunk, 0)
  plsc.subcore_barrier()

  off = 0
  for sz in _sub_pieces(ch):
    pltpu.sync_copy(shared.at[pl.ds(r0 + off, sz)], ybuf.at[pl.ds(0, sz)])
    pltpu.sync_copy(ybuf.at[pl.ds(0, sz)], out_hbm.at[c, pl.ds(r0 + off, sz)])
    off += sz


def _edge_scatter(msg, dst, D):
  ch = 200 if D == 32 else CH
  zeros = jnp.zeros((ch, D), F32)
  k = pl.kernel(
      functools.partial(_scatter_body, D, ch, False),
      out_type=jax.ShapeDtypeStruct((NC, NPAD, D), F32),
      mesh=_MESH,
      scratch_types=[
          pltpu.VMEM_SHARED((NPAD, D), F32),
          pltpu.VMEM((ch,), jnp.int32),
          pltpu.VMEM((ch, D), F32),
      ],
      compiler_params=_SC_PARAMS,
  )
  return k(msg, dst, zeros)


def _edge_counts(dst):
  ones = jnp.ones((CH, 8), F32)
  zeros = jnp.zeros((CH, 8), F32)
  k = pl.kernel(
      functools.partial(_scatter_body, 8, CH, True),
      out_type=jax.ShapeDtypeStruct((NC, NPAD, 8), F32),
      mesh=_MESH,
      scratch_types=[
          pltpu.VMEM_SHARED((NPAD, 8), F32),
          pltpu.VMEM((CH,), jnp.int32),
          pltpu.VMEM((CH, 8), F32),
      ],
      compiler_params=_SC_PARAMS,
  )
  return k(ones, dst, zeros)


def _acc_stats(i, z, dout, stout_ref):
  sums = jnp.sum(z, axis=0)
  sq = jnp.sum(z * z, axis=0)
  pad = 32 - dout
  if pad:
    sums = jnp.concatenate([sums, jnp.zeros((pad,), F32)])
    sq = jnp.concatenate([sq, jnp.zeros((pad,), F32)])
  row = jnp.concatenate([sums, sq])
  upd = jnp.where(lax.broadcasted_iota(jnp.int32, (8, 64), 0) == 0,
                  row[None, :], 0.0)

  @pl.when(i == 0)
  def _():
    stout_ref[...] = jnp.zeros((8, 64), F32)

  stout_ref[...] += upd


def _bn_coeffs(st_ref, g_ref, bt_ref, n):
  st = jnp.sum(st_ref[...], axis=0)
  mean = st[:32] / n
  var = st[32:] / n - mean * mean
  rs = lax.rsqrt(var + EPS)
  return mean, rs, g_ref[0, :], bt_ref[0, :]


def _stats_x_body(z_ref, st_ref):
  i = pl.program_id(0)
  y = z_ref[...]
  _acc_stats(i, y, y.shape[1], st_ref)


def _stats_x(x):
  n, w = x.shape
  return pl.pallas_call(
      _stats_x_body,
      grid=(GRID_X,),
      in_specs=[pl.BlockSpec((BLK_X, w), lambda i: (i, 0))],
      out_specs=pl.BlockSpec((8, 64), lambda i: (0, 0)),
      out_shape=jax.ShapeDtypeStruct((8, 64), F32),
  )(x)


def _passA_body(fin, xi_ref, xj_ref, w_ref, b_ref, out_ref, st_ref):
  i = pl.program_id(0)
  xi = xi_ref[...][:, :fin]
  xj = xj_ref[...][:, :fin]
  e = jnp.concatenate([xi, xj - xi], axis=1)
  z = jnp.dot(e, w_ref[...].T, preferred_element_type=F32) + b_ref[0, :32][None, :]
  z = jnp.maximum(z, 0.0)
  out_ref[...] = z
  _acc_stats(i, z, 32, st_ref)


def _passA(xi, xj, W1, b1, fin):
  wt = xi.shape[1]
  return pl.pallas_call(
      functools.partial(_passA_body, fin),
      grid=(GRID_E,),
      in_specs=[
          pl.BlockSpec((BLK_E, wt), lambda i: (i, 0)),
          pl.BlockSpec((BLK_E, wt), lambda i: (i, 0)),
          pl.BlockSpec((32, 2 * fin), lambda i: (0, 0)),
          pl.BlockSpec((1, 32), lambda i: (0, 0)),
      ],
      out_specs=[
          pl.BlockSpec((BLK_E, 32), lambda i: (i, 0)),
          pl.BlockSpec((8, 64), lambda i: (0, 0)),
      ],
      out_shape=[
          jax.ShapeDtypeStruct((N_EDGES, 32), F32),
          jax.ShapeDtypeStruct((8, 64), F32),
      ],
  )(xi, xj, W1, b1)


def _pass_body(relu_out, with_stats, dout,
               z_ref, st_ref, g_ref, bt_ref, w_ref, b_ref, out_ref, *maybe_st):
  i = pl.program_id(0)
  y = z_ref[...]
  mean, rs, g, bt = _bn_coeffs(st_ref, g_ref, bt_ref, float(N_EDGES))
  yb = (y - mean[None, :]) * rs[None, :] * g[None, :] + bt[None, :]
  z = jnp.dot(yb, w_ref[...].T, preferred_element_type=F32) + b_ref[0, :dout][None, :]
  if relu_out:
    z = jnp.maximum(z, 0.0)
  out_ref[...] = z
  if with_stats:
    _acc_stats(i, z, dout, maybe_st[0])


def _mlp_pass(z, st_prev, gamma, beta, W, b, relu_out, with_stats):
  dout = W.shape[0]
  out_shape = [jax.ShapeDtypeStruct((N_EDGES, dout), F32)]
  out_specs = [pl.BlockSpec((BLK_E, dout), lambda i: (i, 0))]
  if with_stats:
    out_shape.append(jax.ShapeDtypeStruct((8, 64), F32))
    out_specs.append(pl.BlockSpec((8, 64), lambda i: (0, 0)))
  res = pl.pallas_call(
      functools.partial(_pass_body, relu_out, with_stats, dout),
      grid=(GRID_E,),
      in_specs=[
          pl.BlockSpec((BLK_E, 32), lambda i: (i, 0)),
          pl.BlockSpec((8, 64), lambda i: (0, 0)),
          pl.BlockSpec((1, 32), lambda i: (0, 0)),
          pl.BlockSpec((1, 32), lambda i: (0, 0)),
          pl.BlockSpec((dout, 32), lambda i: (0, 0)),
          pl.BlockSpec((1, dout), lambda i: (0, 0)),
      ],
      out_specs=out_specs,
      out_shape=out_shape,
  )(z, st_prev, gamma, beta, W, b)
  return res if with_stats else (res[0], None)


def _prep1_body(x_ref, st_ref, g_ref, bt_ref, h_ref):
  mean, rs, g, bt = _bn_coeffs(st_ref, g_ref, bt_ref, float(N_NODES))
  h = (x_ref[...] - mean[:4][None, :]) * rs[:4][None, :] * g[:4][None, :] + bt[:4][None, :]
  h_ref[...] = jnp.concatenate([h, jnp.zeros((h.shape[0], 4), F32)], axis=1)


def _prep1(x, st0, gamma, beta):
  return pl.pallas_call(
      _prep1_body,
      grid=(GRID_N,),
      in_specs=[
          pl.BlockSpec((BLK_N, 4), lambda i: (i, 0)),
          pl.BlockSpec((8, 64), lambda i: (0, 0)),
          pl.BlockSpec((1, 32), lambda i: (0, 0)),
          pl.BlockSpec((1, 32), lambda i: (0, 0)),
      ],
      out_specs=pl.BlockSpec((BLK_N, 8), lambda i: (i, 0)),
      out_shape=jax.ShapeDtypeStruct((NPAD, 8), F32),
  )(x, st0, gamma, beta)


def _prep_mid_body(fin, wt_out, aggA_ref, aggB_ref, cA_ref, cB_ref, st_ref,
                   g_ref, bt_ref, h_ref):
  cnt = cA_ref[..., 0] + cB_ref[..., 0]
  aggm = (aggA_ref[...] + aggB_ref[...])[:, :fin] / jnp.maximum(cnt, 1.0)[:, None]
  mean, rs, g, bt = _bn_coeffs(st_ref, g_ref, bt_ref, float(N_EDGES))
  h = (aggm - mean[:fin][None, :]) * rs[:fin][None, :] * g[:fin][None, :] + bt[:fin][None, :]
  h = jnp.where((cnt > 0.0)[:, None], h, 0.0)
  pad = wt_out - fin
  if pad:
    h = jnp.concatenate([h, jnp.zeros((h.shape[0], pad), F32)], axis=1)
  h_ref[...] = h


def _prep_mid(agg, cnt2, st3, gamma3, beta3, fin, wt_out):
  dt = agg.shape[-1]
  return pl.pallas_call(
      functools.partial(_prep_mid_body, fin, wt_out),
      grid=(GRID_N,),
      in_specs=[
          pl.BlockSpec((BLK_N, dt), lambda i: (i, 0)),
          pl.BlockSpec((BLK_N, dt), lambda i: (i, 0)),
          pl.BlockSpec((BLK_N, 8), lambda i: (i, 0)),
          pl.BlockSpec((BLK_N, 8), lambda i: (i, 0)),
          pl.BlockSpec((8, 64), lambda i: (0, 0)),
          pl.BlockSpec((1, 32), lambda i: (0, 0)),
          pl.BlockSpec((1, 32), lambda i: (0, 0)),
      ],
      out_specs=pl.BlockSpec((BLK_N, wt_out), lambda i: (i, 0)),
      out_shape=jax.ShapeDtypeStruct((NPAD, wt_out), F32),
  )(agg[0], agg[1], cnt2[0], cnt2[1], st3, gamma3, beta3)


def _post_body(aggA_ref, aggB_ref, cA_ref, cB_ref, out_ref):
  cnt = cA_ref[..., 0] + cB_ref[..., 0]
  out_ref[...] = (aggA_ref[...] + aggB_ref[...])[:, :4] / jnp.maximum(cnt, 1.0)[:, None]


def _post(agg, cnt2):
  return pl.pallas_call(
      _post_body,
      grid=(GRID_N,),
      in_specs=[
          pl.BlockSpec((BLK_N, 8), lambda i: (i, 0)),
          pl.BlockSpec((BLK_N, 8), lambda i: (i, 0)),
          pl.BlockSpec((BLK_N, 8), lambda i: (i, 0)),
          pl.BlockSpec((BLK_N, 8), lambda i: (i, 0)),
      ],
      out_specs=pl.BlockSpec((BLK_N, 4), lambda i: (i, 0)),
      out_shape=jax.ShapeDtypeStruct((NPAD, 4), F32),
  )(agg[0], agg[1], cnt2[0], cnt2[1])


def _pad32(v):
  return jnp.pad(v, (0, 32 - v.shape[0])).reshape(1, 32)


def kernel(x, edge_index, params):
  src = edge_index[0]
  dst = edge_index[1]

  cnt2 = _edge_counts(dst)

  convs = [
      ("enc1", 4, 32, 32, True),
      ("enc2", 32, 32, 32, True),
      ("enc3", 32, 2, 8, True),
      ("dec1", 2, 32, 32, True),
      ("dec2", 32, 32, 32, True),
      ("dec3", 32, 4, 8, False),
  ]

  st0 = _stats_x(x)
  h = _prep1(x, st0, _pad32(params["bn0"]["gamma"]), _pad32(params["bn0"]["beta"]))

  gather8 = _edge_gather(jax.ShapeDtypeStruct((NPAD, 8), F32))
  gather32 = _edge_gather(jax.ShapeDtypeStruct((NPAD, 32), F32))

  for ci, (name, fin, fout, dpad, fa) in enumerate(convs):
    layers = params[name]
    l1, l2, l3 = layers

    gather = gather8 if h.shape[1] == 8 else gather32
    XI, XJ = gather(h, src, dst)
    Y1, st1 = _passA(XI, XJ, l1["lin"]["W"], _pad32(l1["lin"]["b"]), fin)
    Y2, st2 = _mlp_pass(Y1, st1, _pad32(l1["bn"]["gamma"]), _pad32(l1["bn"]["beta"]),
                        l2["lin"]["W"], _pad32(l2["lin"]["b"])[:, :32],
                        relu_out=True, with_stats=True)
    W3 = l3["lin"]["W"]
    b3 = l3["lin"]["b"]
    if dpad != W3.shape[0]:
      W3 = jnp.pad(W3, ((0, dpad - W3.shape[0]), (0, 0)))
      b3 = jnp.pad(b3, (0, dpad - b3.shape[0]))
    b3 = b3.reshape(1, dpad)
    Y3, st3 = _mlp_pass(Y2, st2, _pad32(l2["bn"]["gamma"]), _pad32(l2["bn"]["beta"]),
                        W3, b3, relu_out=fa, with_stats=fa)
    agg = _edge_scatter(Y3, dst, dpad)

    if ci < 5:
      wt_next = 8 if fout < 8 else 32
      h = _prep_mid(agg, cnt2, st3, _pad32(l3["bn"]["gamma"]),
                    _pad32(l3["bn"]["beta"]), fout, wt_next)
    else:
      out = _post(agg, cnt2)

  return out[:N_NODES]


# speedup vs baseline: 2.7624x; 1.0000x over previous
import functools

import jax
import jax.numpy as jnp
from jax import lax
from jax.experimental import pallas as pl
from jax.experimental.pallas import tpu as pltpu
from jax.experimental.pallas import tpu_sc as plsc

N_NODES = 50000
N_EDGES = 800000
EPS = 1e-5
NPAD = 50048

NC, NS = 2, 16
NW = NC * NS
EPT = N_EDGES // NW
CH = 1000
NCH = EPT // CH
RZT = NPAD // NS


def _sub_pieces(ch):
  pieces = [ch] * (RZT // ch)
  if RZT % ch:
    pieces.append(RZT % ch)
  return tuple(pieces)

BLK_E = 8000
GRID_E = N_EDGES // BLK_E
BLK_N = 6256
GRID_N = NPAD // BLK_N
BLK_X = 10000
GRID_X = N_NODES // BLK_X

_MESH = plsc.VectorSubcoreMesh(core_axis_name="c", subcore_axis_name="s")
_SC_PARAMS = pltpu.CompilerParams(use_tc_tiling_on_sc=False)
F32 = jnp.float32


def _gather_body(h_hbm, src_hbm, dst_hbm, xi_hbm, xj_hbm, idx_d, idx_s,
                 bi, bj, sem):
  wid = lax.axis_index("s") * NC + lax.axis_index("c")
  base_t = wid * EPT

  def chunk(k, carry):
    base = base_t + k * CH
    pltpu.sync_copy(dst_hbm.at[pl.ds(base, CH)], idx_d)
    pltpu.sync_copy(src_hbm.at[pl.ds(base, CH)], idx_s)
    pltpu.async_copy(h_hbm.at[idx_d], bi, sem).wait()
    pltpu.async_copy(h_hbm.at[idx_s], bj, sem).wait()
    pltpu.sync_copy(bi, xi_hbm.at[pl.ds(base, CH)])
    pltpu.sync_copy(bj, xj_hbm.at[pl.ds(base, CH)])
    return carry

  lax.fori_loop(0, NCH, chunk, 0)


def _edge_gather(h):
  w = h.shape[1]
  k = pl.kernel(
      _gather_body,
      out_type=[
          jax.ShapeDtypeStruct((N_EDGES, w), F32),
          jax.ShapeDtypeStruct((N_EDGES, w), F32),
      ],
      mesh=_MESH,
      scratch_types=[
          pltpu.VMEM((CH,), jnp.int32),
          pltpu.VMEM((CH,), jnp.int32),
          pltpu.VMEM((CH, w), F32),
          pltpu.VMEM((CH, w), F32),
          pltpu.SemaphoreType.DMA,
      ],
      compiler_params=_SC_PARAMS,
  )
  return k


def _scatter_body(D, ch, counts, msg_hbm, dst_hbm, zeros_hbm, out_hbm, shared,
                  idx_v, ybuf):
  c = lax.axis_index("c")
  s = lax.axis_index("s")
  wid = s * NC + c
  r0 = s * RZT
  nch = EPT // ch

  pltpu.sync_copy(zeros_hbm, ybuf)
  off = 0
  for sz in _sub_pieces(ch):
    pltpu.sync_copy(ybuf.at[pl.ds(0, sz)], shared.at[pl.ds(r0 + off, sz)])
    off += sz
  plsc.subcore_barrier()

  if counts:
    pltpu.sync_copy(msg_hbm, ybuf)

    def chunk(k, carry):
      base = wid * EPT + k * ch
      pltpu.sync_copy(dst_hbm.at[pl.ds(base, ch)], idx_v)
      pltpu.sync_copy(ybuf, shared.at[idx_v], add=True)
      return carry
  else:

    def chunk(k, carry):
      base = wid * EPT + k * ch
      pltpu.sync_copy(dst_hbm.at[pl.ds(base, ch)], idx_v)
      pltpu.sync_copy(msg_hbm.at[pl.ds(base, ch)], ybuf)
      pltpu.sync_copy(ybuf, shared.at[idx_v], add=True)
      return carry

  lax.fori_loop(0, nch, chunk, 0)
  plsc.subcore_barrier()

  off = 0
  for sz in _sub_pieces(ch):
    pltpu.sync_copy(shared.at[pl.ds(r0 + off, sz)], ybuf.at[pl.ds(0, sz)])
    pltpu.sync_copy(ybuf.at[pl.ds(0, sz)], out_hbm.at[c, pl.ds(r0 + off, sz)])
    off += sz


def _edge_scatter(msg, dst, D):
  ch = 200 if D == 32 else CH
  zeros = jnp.zeros((ch, D), F32)
  k = pl.kernel(
      functools.partial(_scatter_body, D, ch, False),
      out_type=jax.ShapeDtypeStruct((NC, NPAD, D), F32),
      mesh=_MESH,
      scratch_types=[
          pltpu.VMEM_SHARED((NPAD, D), F32),
          pltpu.VMEM((ch,), jnp.int32),
          pltpu.VMEM((ch, D), F32),
      ],
      compiler_params=_SC_PARAMS,
  )
  return k(msg, dst, zeros)


def _edge_counts(dst):
  ones = jnp.ones((CH, 8), F32)
  zeros = jnp.zeros((CH, 8), F32)
  k = pl.kernel(
      functools.partial(_scatter_body, 8, CH, True),
      out_type=jax.ShapeDtypeStruct((NC, NPAD, 8), F32),
      mesh=_MESH,
      scratch_types=[
          pltpu.VMEM_SHARED((NPAD, 8), F32),
          pltpu.VMEM((CH,), jnp.int32),
          pltpu.VMEM((CH, 8), F32),
      ],
      compiler_params=_SC_PARAMS,
  )
  return k(ones, dst, zeros)


def _acc_stats(i, z, dout, stout_ref):
  sums = jnp.sum(z, axis=0)
  sq = jnp.sum(z * z, axis=0)
  pad = 32 - dout
  if pad:
    sums = jnp.concatenate([sums, jnp.zeros((pad,), F32)])
    sq = jnp.concatenate([sq, jnp.zeros((pad,), F32)])
  row = jnp.concatenate([sums, sq])
  upd = jnp.where(lax.broadcasted_iota(jnp.int32, (8, 64), 0) == 0,
                  row[None, :], 0.0)

  @pl.when(i == 0)
  def _():
    stout_ref[...] = jnp.zeros((8, 64), F32)

  stout_ref[...] += upd


def _bn_coeffs(st_ref, g_ref, bt_ref, n):
  st = jnp.sum(st_ref[...], axis=0)
  mean = st[:32] / n
  var = st[32:] / n - mean * mean
  rs = lax.rsqrt(var + EPS)
  return mean, rs, g_ref[0, :], bt_ref[0, :]


def _stats_x_body(z_ref, st_ref):
  i = pl.program_id(0)
  y = z_ref[...]
  _acc_stats(i, y, y.shape[1], st_ref)


def _stats_x(x):
  n, w = x.shape
  return pl.pallas_call(
      _stats_x_body,
      grid=(GRID_X,),
      in_specs=[pl.BlockSpec((BLK_X, w), lambda i: (i, 0))],
      out_specs=pl.BlockSpec((8, 64), lambda i: (0, 0)),
      out_shape=jax.ShapeDtypeStruct((8, 64), F32),
  )(x)


def _passA_body(fin, xi_ref, xj_ref, w_ref, b_ref, out_ref, st_ref):
  i = pl.program_id(0)
  xi = xi_ref[...][:, :fin]
  xj = xj_ref[...][:, :fin]
  e = jnp.concatenate([xi, xj - xi], axis=1)
  z = jnp.dot(e, w_ref[...].T, preferred_element_type=F32) + b_ref[0, :32][None, :]
  z = jnp.maximum(z, 0.0)
  out_ref[...] = z
  _acc_stats(i, z, 32, st_ref)


def _passA(xi, xj, W1, b1, fin):
  wt = xi.shape[1]
  return pl.pallas_call(
      functools.partial(_passA_body, fin),
      grid=(GRID_E,),
      in_specs=[
          pl.BlockSpec((BLK_E, wt), lambda i: (i, 0)),
          pl.BlockSpec((BLK_E, wt), lambda i: (i, 0)),
          pl.BlockSpec((32, 2 * fin), lambda i: (0, 0)),
          pl.BlockSpec((1, 32), lambda i: (0, 0)),
      ],
      out_specs=[
          pl.BlockSpec((BLK_E, 32), lambda i: (i, 0)),
          pl.BlockSpec((8, 64), lambda i: (0, 0)),
      ],
      out_shape=[
          jax.ShapeDtypeStruct((N_EDGES, 32), F32),
          jax.ShapeDtypeStruct((8, 64), F32),
      ],
  )(xi, xj, W1, b1)


def _pass_body(relu_out, with_stats, dout,
               z_ref, st_ref, g_ref, bt_ref, w_ref, b_ref, out_ref, *maybe_st):
  i = pl.program_id(0)
  y = z_ref[...]
  mean, rs, g, bt = _bn_coeffs(st_ref, g_ref, bt_ref, float(N_EDGES))
  yb = (y - mean[None, :]) * rs[None, :] * g[None, :] + bt[None, :]
  z = jnp.dot(yb, w_ref[...].T, preferred_element_type=F32) + b_ref[0, :dout][None, :]
  if relu_out:
    z = jnp.maximum(z, 0.0)
  out_ref[...] = z
  if with_stats:
    _acc_stats(i, z, dout, maybe_st[0])


def _mlp_pass(z, st_prev, gamma, beta, W, b, relu_out, with_stats):
  dout = W.shape[0]
  out_shape = [jax.ShapeDtypeStruct((N_EDGES, dout), F32)]
  out_specs = [pl.BlockSpec((BLK_E, dout), lambda i: (i, 0))]
  if with_stats:
    out_shape.append(jax.ShapeDtypeStruct((8, 64), F32))
    out_specs.append(pl.BlockSpec((8, 64), lambda i: (0, 0)))
  res = pl.pallas_call(
      functools.partial(_pass_body, relu_out, with_stats, dout),
      grid=(GRID_E,),
      in_specs=[
          pl.BlockSpec((BLK_E, 32), lambda i: (i, 0)),
          pl.BlockSpec((8, 64), lambda i: (0, 0)),
          pl.BlockSpec((1, 32), lambda i: (0, 0)),
          pl.BlockSpec((1, 32), lambda i: (0, 0)),
          pl.BlockSpec((dout, 32), lambda i: (0, 0)),
          pl.BlockSpec((1, dout), lambda i: (0, 0)),
      ],
      out_specs=out_specs,
      out_shape=out_shape,
  )(z, st_prev, gamma, beta, W, b)
  return res if with_stats else (res[0], None)


def _prep1_body(x_ref, st_ref, g_ref, bt_ref, h_ref):
  mean, rs, g, bt = _bn_coeffs(st_ref, g_ref, bt_ref, float(N_NODES))
  h = (x_ref[...] - mean[:4][None, :]) * rs[:4][None, :] * g[:4][None, :] + bt[:4][None, :]
  h_ref[...] = jnp.concatenate([h, jnp.zeros((h.shape[0], 4), F32)], axis=1)


def _prep1(x, st0, gamma, beta):
  return pl.pallas_call(
      _prep1_body,
      grid=(GRID_N,),
      in_specs=[
          pl.BlockSpec((BLK_N, 4), lambda i: (i, 0)),
          pl.BlockSpec((8, 64), lambda i: (0, 0)),
          pl.BlockSpec((1, 32), lambda i: (0, 0)),
          pl.BlockSpec((1, 32), lambda i: (0, 0)),
      ],
      out_specs=pl.BlockSpec((BLK_N, 8), lambda i: (i, 0)),
      out_shape=jax.ShapeDtypeStruct((NPAD, 8), F32),
  )(x, st0, gamma, beta)


def _prep_mid_body(fin, wt_out, aggA_ref, aggB_ref, cA_ref, cB_ref, st_ref,
                   g_ref, bt_ref, h_ref):
  cnt = cA_ref[..., 0] + cB_ref[..., 0]
  aggm = (aggA_ref[...] + aggB_ref[...])[:, :fin] / jnp.maximum(cnt, 1.0)[:, None]
  mean, rs, g, bt = _bn_coeffs(st_ref, g_ref, bt_ref, float(N_EDGES))
  h = (aggm - mean[:fin][None, :]) * rs[:fin][None, :] * g[:fin][None, :] + bt[:fin][None, :]
  h = jnp.where((cnt > 0.0)[:, None], h, 0.0)
  pad = wt_out - fin
  if pad:
    h = jnp.concatenate([h, jnp.zeros((h.shape[0], pad), F32)], axis=1)
  h_ref[...] = h


def _prep_mid(agg, cnt2, st3, gamma3, beta3, fin, wt_out):
  dt = agg.shape[-1]
  return pl.pallas_call(
      functools.partial(_prep_mid_body, fin, wt_out),
      grid=(GRID_N,),
      in_specs=[
          pl.BlockSpec((BLK_N, dt), lambda i: (i, 0)),
          pl.BlockSpec((BLK_N, dt), lambda i: (i, 0)),
          pl.BlockSpec((BLK_N, 8), lambda i: (i, 0)),
          pl.BlockSpec((BLK_N, 8), lambda i: (i, 0)),
          pl.BlockSpec((8, 64), lambda i: (0, 0)),
          pl.BlockSpec((1, 32), lambda i: (0, 0)),
          pl.BlockSpec((1, 32), lambda i: (0, 0)),
      ],
      out_specs=pl.BlockSpec((BLK_N, wt_out), lambda i: (i, 0)),
      out_shape=jax.ShapeDtypeStruct((NPAD, wt_out), F32),
  )(agg[0], agg[1], cnt2[0], cnt2[1], st3, gamma3, beta3)


def _post_body(aggA_ref, aggB_ref, cA_ref, cB_ref, out_ref):
  cnt = cA_ref[..., 0] + cB_ref[..., 0]
  out_ref[...] = (aggA_ref[...] + aggB_ref[...])[:, :4] / jnp.maximum(cnt, 1.0)[:, None]


def _post(agg, cnt2):
  return pl.pallas_call(
      _post_body,
      grid=(GRID_N,),
      in_specs=[
          pl.BlockSpec((BLK_N, 8), lambda i: (i, 0)),
          pl.BlockSpec((BLK_N, 8), lambda i: (i, 0)),
          pl.BlockSpec((BLK_N, 8), lambda i: (i, 0)),
          pl.BlockSpec((BLK_N, 8), lambda i: (i, 0)),
      ],
      out_specs=pl.BlockSpec((BLK_N, 4), lambda i: (i, 0)),
      out_shape=jax.ShapeDtypeStruct((NPAD, 4), F32),
  )(agg[0], agg[1], cnt2[0], cnt2[1])


def _pad32(v):
  return jnp.pad(v, (0, 32 - v.shape[0])).reshape(1, 32)


def kernel(x, edge_index, params):
  src = edge_index[0]
  dst = edge_index[1]

  cnt2 = _edge_counts(dst)
  dep = jnp.int32(0) * cnt2[0, 0, 0].astype(jnp.int32)
  src = src + dep
  dst = dst + dep

  convs = [
      ("enc1", 4, 32, 32, True),
      ("enc2", 32, 32, 32, True),
      ("enc3", 32, 2, 8, True),
      ("dec1", 2, 32, 32, True),
      ("dec2", 32, 32, 32, True),
      ("dec3", 32, 4, 8, False),
  ]

  st0 = _stats_x(x)
  h = _prep1(x, st0, _pad32(params["bn0"]["gamma"]), _pad32(params["bn0"]["beta"]))

  gather8 = _edge_gather(jax.ShapeDtypeStruct((NPAD, 8), F32))
  gather32 = _edge_gather(jax.ShapeDtypeStruct((NPAD, 32), F32))

  for ci, (name, fin, fout, dpad, fa) in enumerate(convs):
    layers = params[name]
    l1, l2, l3 = layers

    gather = gather8 if h.shape[1] == 8 else gather32
    XI, XJ = gather(h, src, dst)
    Y1, st1 = _passA(XI, XJ, l1["lin"]["W"], _pad32(l1["lin"]["b"]), fin)
    Y2, st2 = _mlp_pass(Y1, st1, _pad32(l1["bn"]["gamma"]), _pad32(l1["bn"]["beta"]),
                        l2["lin"]["W"], _pad32(l2["lin"]["b"])[:, :32],
                        relu_out=True, with_stats=True)
    W3 = l3["lin"]["W"]
    b3 = l3["lin"]["b"]
    if dpad != W3.shape[0]:
      W3 = jnp.pad(W3, ((0, dpad - W3.shape[0]), (0, 0)))
      b3 = jnp.pad(b3, (0, dpad - b3.shape[0]))
    b3 = b3.reshape(1, dpad)
    Y3, st3 = _mlp_pass(Y2, st2, _pad32(l2["bn"]["gamma"]), _pad32(l2["bn"]["beta"]),
                        W3, b3, relu_out=fa, with_stats=fa)
    agg = _edge_scatter(Y3, dst, dpad)

    if ci < 5:
      wt_next = 8 if fout < 8 else 32
      h = _prep_mid(agg, cnt2, st3, _pad32(l3["bn"]["gamma"]),
                    _pad32(l3["bn"]["beta"]), fout, wt_next)
    else:
      out = _post(agg, cnt2)

  return out[:N_NODES]
